# baseline (speedup 1.0000x reference)
import jax
import jax.numpy as jnp
from jax.experimental import pallas as pl

NUM_NODES = 100


def _gat_conv(x, src, dst, W, att_s, att_d, bias, heads, out_ch, concat):
    N = x.shape[0]
    loops = jnp.arange(N, dtype=src.dtype)
    s = jnp.concatenate([src, loops])
    d = jnp.concatenate([dst, loops])
    h = (x @ W).reshape(N, heads, out_ch)
    a_s = (h * att_s[None]).sum(-1)
    a_d = (h * att_d[None]).sum(-1)
    e = jax.nn.leaky_relu(a_s[s] + a_d[d], 0.2)
    e_max = jax.ops.segment_max(e, d, num_segments=N)
    e_max = jnp.where(jnp.isfinite(e_max), e_max, 0.0)
    ex = jnp.exp(e - e_max[d])
    den = jax.ops.segment_sum(ex, d, num_segments=N)
    alpha = ex / (den[d] + 1e-16)
    out = jax.ops.segment_sum(h[s] * alpha[..., None], d, num_segments=N)
    if concat:
        out = out.reshape(N, heads * out_ch)
    else:
        out = out.mean(axis=1)
    return out + bias


def _head_matmul_kernel(h_ref, w_ref, b_ref, o_ref):
    h = h_ref[0]
    w = w_ref[0]
    o_ref[0] = jnp.dot(h, w, preferred_element_type=jnp.float32) + b_ref[0]


def kernel(x, edge_index, emb_tables, W1, att_src1, att_dst1, b1, W2,
           att_src2, att_dst2, b2, head_W, head_b):
    bsz = x.shape[0] // NUM_NODES
    xv = jnp.clip(x.reshape(bsz, NUM_NODES), 0, 16)
    emb = emb_tables[jnp.arange(NUM_NODES)[None, :], xv]
    h = emb.reshape(bsz * NUM_NODES, 16)
    src, dst = edge_index[0], edge_index[1]
    h = _gat_conv(h, src, dst, W1, att_src1, att_dst1, b1, 4, 16, True)
    h = jax.nn.relu(h)
    h = _gat_conv(h, src, dst, W2, att_src2, att_dst2, b2, 1, 16, False)
    h = h.reshape(bsz, NUM_NODES, 16)
    hT = h.transpose(1, 0, 2)
    out = pl.pallas_call(
        _head_matmul_kernel,
        grid=(NUM_NODES,),
        in_specs=[
            pl.BlockSpec((1, bsz, 16), lambda n: (n, 0, 0)),
            pl.BlockSpec((1, 16, 16), lambda n: (n, 0, 0)),
            pl.BlockSpec((1, 1, 16), lambda n: (n, 0, 0)),
        ],
        out_specs=pl.BlockSpec((1, bsz, 16), lambda n: (n, 0, 0)),
        out_shape=jax.ShapeDtypeStruct((NUM_NODES, bsz, 16), jnp.float32),
    )(hT, head_W, head_b.reshape(NUM_NODES, 1, 16))
    return out.transpose(1, 0, 2)


# speedup vs baseline: 23.9326x; 23.9326x over previous
import jax
import jax.numpy as jnp
from jax import lax
from jax.experimental import pallas as pl
from jax.experimental.pallas import tpu as pltpu
from jax.experimental.pallas import tpu_sc as plsc

NUM_NODES = 100
NV = 17
NCLS = NUM_NODES * NV
NCLSP = 1792
N = 102400
E = 1228800
NC, NS = 2, 16

E2 = E // NC
EPT = E2 // NS
KB = 128
NBLK = EPT // KB
HALF = N // NC
NPTH = HALF // NS
SB = 128
NPT = N // NS
ZB = 128


def _lrelu_exp(z):
    return jnp.exp(jnp.where(z >= 0, z, z * 0.2))


def _tables_kernel(emb_ref, w1_ref, as_ref, ad_ref, ct_ref):
    tb = jnp.dot(emb_ref[...], w1_ref[...], preferred_element_type=jnp.float32)
    cols = []
    for h in range(4):
        tbh = tb[:, 16 * h:16 * h + 16]
        a_s = jnp.sum(tbh * as_ref[h:h + 1, :], axis=1, keepdims=True)
        a_d = jnp.sum(tbh * ad_ref[h:h + 1, :], axis=1, keepdims=True)
        cols += [tbh, a_s, a_d]
    ct = jnp.concatenate(cols, axis=1)
    ct_ref[...] = jnp.concatenate(
        [ct, jnp.zeros((NCLSP - NCLS, 72), jnp.float32)], axis=0)


def _expand_kernel(x_ref, ct_ref, *out_refs):
    nb = x_ref.shape[0]
    gid = pl.program_id(0)
    ids = gid * nb + jax.lax.broadcasted_iota(jnp.int32, (nb, 1), 0)
    typ = ids - (ids // NUM_NODES) * NUM_NODES
    xv = jnp.clip(x_ref[...], 0, 16).astype(jnp.int32)
    cls = typ * NV + xv
    onehot = (cls == jax.lax.broadcasted_iota(jnp.int32, (nb, NCLSP), 1))
    res = jnp.dot(onehot.astype(jnp.float32), ct_ref[...],
                  preferred_element_type=jnp.float32)
    ones = jnp.ones((1, 16), jnp.float32)
    ones4 = jnp.ones((1, 4), jnp.float32)
    as4, ad4 = [], []
    for h in range(4):
        feat = res[:, 18 * h:18 * h + 16]
        asc = res[:, 18 * h + 16:18 * h + 17]
        adc = res[:, 18 * h + 17:18 * h + 18]
        out_refs[h][...] = jnp.concatenate([feat, asc * ones], axis=1)
        out_refs[4 + h][...] = adc * ones
        as4.append(asc * ones4)
        ad4.append(adc * ones4)
    out_refs[8][...] = jnp.concatenate(as4, axis=1)
    out_refs[9][...] = jnp.concatenate(ad4, axis=1)


def _zero_acc(acc_sh, outb, base):
    z16 = jnp.zeros((16,), jnp.float32)

    @pl.loop(0, ZB)
    def _(i):
        outb[i, pl.ds(0, 16)] = z16

    @pl.loop(0, NPT // ZB)
    def _(i):
        pltpu.sync_copy(outb, acc_sh.at[pl.ds(base + i * ZB, ZB)])


def _feat_edge_pass(src_hbm, dst_hbm, fa_hbm, ad_hbm, acc_sh,
                    srcb, dstb, fab, adrows, outb, eoff):
    @pl.loop(0, NBLK)
    def _(b):
        off = eoff + b * KB
        pltpu.sync_copy(src_hbm.at[pl.ds(off, KB)], srcb)
        pltpu.sync_copy(dst_hbm.at[pl.ds(off, KB)], dstb)
        pltpu.sync_copy(fa_hbm.at[srcb], fab)
        pltpu.sync_copy(ad_hbm.at[dstb], adrows)

        @pl.loop(0, KB, unroll=8)
        def _(r):
            w = _lrelu_exp(fab[r, pl.ds(16, 16)] + adrows[r, pl.ds(0, 16)])
            outb[r, pl.ds(0, 16)] = fab[r, pl.ds(0, 16)] * w

        pltpu.sync_copy(outb, acc_sh.at[dstb], add=True)


def _feat_self_pass(fa_hbm, ad_hbm, acc_sh, fab, adrows, outb, idxb, noff):
    @pl.loop(0, NPTH // SB)
    def _(b):
        goff = noff + b * SB
        pltpu.sync_copy(fa_hbm.at[pl.ds(goff, SB)], fab)
        pltpu.sync_copy(ad_hbm.at[pl.ds(goff, SB)], adrows)

        @pl.loop(0, SB // 16)
        def _(g):
            idxb[pl.ds(g * 16, 16)] = (jnp.arange(16, dtype=jnp.int32)
                                       + (goff + g * 16))

        @pl.loop(0, SB, unroll=8)
        def _(r):
            w = _lrelu_exp(fab[r, pl.ds(16, 16)] + adrows[r, pl.ds(0, 16)])
            outb[r, pl.ds(0, 16)] = fab[r, pl.ds(0, 16)] * w

        pltpu.sync_copy(outb, acc_sh.at[idxb], add=True)


def _den_edge_pass(src_hbm, dst_hbm, as_hbm, ad_hbm, acc_sh,
                   srcb, dstb, asrows, adrows, outb, eoff):
    @pl.loop(0, NBLK)
    def _(b):
        off = eoff + b * KB
        pltpu.sync_copy(src_hbm.at[pl.ds(off, KB)], srcb)
        pltpu.sync_copy(dst_hbm.at[pl.ds(off, KB)], dstb)
        pltpu.sync_copy(as_hbm.at[srcb], asrows)
        pltpu.sync_copy(ad_hbm.at[dstb], adrows)

        @pl.loop(0, KB, unroll=8)
        def _(r):
            outb[r, pl.ds(0, 16)] = _lrelu_exp(asrows[r, pl.ds(0, 16)]
                                               + adrows[r, pl.ds(0, 16)])

        pltpu.sync_copy(outb, acc_sh.at[dstb], add=True)


def _den_self_pass(as_hbm, ad_hbm, acc_sh, asrows, adrows, outb, idxb, noff):
    @pl.loop(0, NPTH // SB)
    def _(b):
        goff = noff + b * SB
        pltpu.sync_copy(as_hbm.at[pl.ds(goff, SB)], asrows)
        pltpu.sync_copy(ad_hbm.at[pl.ds(goff, SB)], adrows)

        @pl.loop(0, SB // 16)
        def _(g):
            idxb[pl.ds(g * 16, 16)] = (jnp.arange(16, dtype=jnp.int32)
                                       + (goff + g * 16))

        @pl.loop(0, SB, unroll=8)
        def _(r):
            outb[r, pl.ds(0, 16)] = _lrelu_exp(asrows[r, pl.ds(0, 16)]
                                               + adrows[r, pl.ds(0, 16)])

        pltpu.sync_copy(outb, acc_sh.at[idxb], add=True)


def _writeback(acc_sh, acc_hbm, sid, slot_off):
    pltpu.sync_copy(acc_sh.at[pl.ds(sid * NPT, NPT)],
                    acc_hbm.at[pl.ds(slot_off + sid * NPT, NPT)])


def _sc1_body(src_hbm, dst_hbm, fa0, fa1, fa2, fa3, ad0, ad1, ad2, ad3,
              as4_hbm, ad4_hbm, acc_hbm,
              acc_sh, srcb, dstb, fab, adrows, asrows, outb, idxb):
    sid = lax.axis_index("s")
    cid = lax.axis_index("c")
    fas = [fa0, fa1, fa2, fa3]
    ads = [ad0, ad1, ad2, ad3]
    eoff = cid * E2 + sid * EPT
    noff = cid * HALF + sid * NPTH
    for p in range(4):
        _zero_acc(acc_sh, outb, sid * NPT)
        plsc.subcore_barrier()
        _feat_edge_pass(src_hbm, dst_hbm, fas[p], ads[p], acc_sh,
                        srcb, dstb, fab, adrows, outb, eoff)
        _feat_self_pass(fas[p], ads[p], acc_sh, fab, adrows, outb, idxb,
                        noff)
        plsc.subcore_barrier()
        _writeback(acc_sh, acc_hbm, sid, cid * 5 * N + p * N)
        plsc.subcore_barrier()
    _zero_acc(acc_sh, outb, sid * NPT)
    plsc.subcore_barrier()
    _den_edge_pass(src_hbm, dst_hbm, as4_hbm, ad4_hbm, acc_sh,
                   srcb, dstb, asrows, adrows, outb, eoff)
    _den_self_pass(as4_hbm, ad4_hbm, acc_sh, asrows, adrows, outb, idxb,
                   noff)
    plsc.subcore_barrier()
    _writeback(acc_sh, acc_hbm, sid, cid * 5 * N + 4 * N)


def _sc2_body(src_hbm, dst_hbm, fa_hbm, ad_hbm, as2_hbm, acc_hbm,
              acc_sh, srcb, dstb, fab, adrows, asrows, outb, idxb):
    sid = lax.axis_index("s")
    cid = lax.axis_index("c")
    eoff = cid * E2 + sid * EPT
    noff = cid * HALF + sid * NPTH
    _zero_acc(acc_sh, outb, sid * NPT)
    plsc.subcore_barrier()
    _feat_edge_pass(src_hbm, dst_hbm, fa_hbm, ad_hbm, acc_sh,
                    srcb, dstb, fab, adrows, outb, eoff)
    _feat_self_pass(fa_hbm, ad_hbm, acc_sh, fab, adrows, outb, idxb, noff)
    plsc.subcore_barrier()
    _writeback(acc_sh, acc_hbm, sid, cid * 2 * N)
    plsc.subcore_barrier()
    _zero_acc(acc_sh, outb, sid * NPT)
    plsc.subcore_barrier()
    _den_edge_pass(src_hbm, dst_hbm, as2_hbm, ad_hbm, acc_sh,
                   srcb, dstb, asrows, adrows, outb, eoff)
    _den_self_pass(as2_hbm, ad_hbm, acc_sh, asrows, adrows, outb, idxb,
                   noff)
    plsc.subcore_barrier()
    _writeback(acc_sh, acc_hbm, sid, cid * 2 * N + N)


def _mid_kernel(acc_ref, w2_ref, as2_ref, ad2_ref, b1_ref,
                fa_ref, ad_ref, as_ref):
    den = acc_ref[4] + acc_ref[9]
    outs = []
    for h in range(4):
        f = acc_ref[h] + acc_ref[5 + h]
        outs.append(f / (den[:, 4 * h:4 * h + 1] + 1e-16))
    x2 = jnp.maximum(jnp.concatenate(outs, axis=1) + b1_ref[...], 0.0)
    h2 = jnp.dot(x2, w2_ref[...], preferred_element_type=jnp.float32)
    ones = jnp.ones((1, 16), jnp.float32)
    a_s2 = jnp.sum(h2 * as2_ref[...], axis=1, keepdims=True)
    a_d2 = jnp.sum(h2 * ad2_ref[...], axis=1, keepdims=True)
    fa_ref[...] = jnp.concatenate([h2, a_s2 * ones], axis=1)
    ad_ref[...] = a_d2 * ones
    as_ref[...] = a_s2 * ones


def _head_kernel(acc_ref, b2_ref, w_ref, hb_ref, o_ref):
    f = acc_ref[0, 0] + acc_ref[0, 2]
    den = acc_ref[0, 1] + acc_ref[0, 3]
    hf = f / (den[:, 0:1] + 1e-16) + b2_ref[...]
    o_ref[0] = (jnp.dot(hf, w_ref[0], preferred_element_type=jnp.float32)
                + hb_ref[0])


def kernel(x, edge_index, emb_tables, W1, att_src1, att_dst1, b1, W2,
           att_src2, att_dst2, b2, head_W, head_b):
    bsz = x.shape[0] // NUM_NODES
    src = edge_index[0].astype(jnp.int32)
    dst = edge_index[1].astype(jnp.int32)

    ct = pl.pallas_call(
        _tables_kernel,
        out_shape=jax.ShapeDtypeStruct((NCLSP, 72), jnp.float32),
    )(emb_tables.reshape(NCLS, 16), W1, att_src1, att_dst1)

    nb = 512
    outs = pl.pallas_call(
        _expand_kernel,
        grid=(N // nb,),
        in_specs=[
            pl.BlockSpec((nb, 1), lambda g: (g, 0)),
            pl.BlockSpec((NCLSP, 72), lambda g: (0, 0)),
        ],
        out_specs=tuple([pl.BlockSpec((nb, 32), lambda g: (g, 0))] * 4
                        + [pl.BlockSpec((nb, 16), lambda g: (g, 0))] * 6),
        out_shape=tuple([jax.ShapeDtypeStruct((N, 32), jnp.float32)] * 4
                        + [jax.ShapeDtypeStruct((N, 16), jnp.float32)] * 6),
    )(x, ct)
    fa_tabs, ad_tabs, as4, ad4 = outs[:4], outs[4:8], outs[8], outs[9]

    mesh = plsc.VectorSubcoreMesh(core_axis_name="c", subcore_axis_name="s",
                                  num_cores=NC, num_subcores=NS)
    sc_scratch = [
        pltpu.VMEM_SHARED((N, 16), jnp.float32),
        pltpu.VMEM((KB,), jnp.int32),
        pltpu.VMEM((KB,), jnp.int32),
        pltpu.VMEM((KB, 32), jnp.float32),
        pltpu.VMEM((KB, 16), jnp.float32),
        pltpu.VMEM((KB, 16), jnp.float32),
        pltpu.VMEM((KB, 16), jnp.float32),
        pltpu.VMEM((KB,), jnp.int32),
    ]

    sc1 = pl.kernel(
        _sc1_body,
        out_type=jax.ShapeDtypeStruct((2 * 5 * N, 16), jnp.float32),
        mesh=mesh,
        compiler_params=pltpu.CompilerParams(use_tc_tiling_on_sc=False),
        scratch_types=sc_scratch,
    )
    acc1 = sc1(src, dst, *fa_tabs, *ad_tabs, as4, ad4)

    t2fa, tad2, tas2 = pl.pallas_call(
        _mid_kernel,
        grid=(N // 1024,),
        in_specs=[
            pl.BlockSpec((10, 1024, 16), lambda g: (0, g, 0)),
            pl.BlockSpec((64, 16), lambda g: (0, 0)),
            pl.BlockSpec((1, 16), lambda g: (0, 0)),
            pl.BlockSpec((1, 16), lambda g: (0, 0)),
            pl.BlockSpec((1, 64), lambda g: (0, 0)),
        ],
        out_specs=(pl.BlockSpec((1024, 32), lambda g: (g, 0)),
                   pl.BlockSpec((1024, 16), lambda g: (g, 0)),
                   pl.BlockSpec((1024, 16), lambda g: (g, 0))),
        out_shape=(jax.ShapeDtypeStruct((N, 32), jnp.float32),
                   jax.ShapeDtypeStruct((N, 16), jnp.float32),
                   jax.ShapeDtypeStruct((N, 16), jnp.float32)),
    )(acc1.reshape(2, 5, N, 16).reshape(10, N, 16), W2, att_src2, att_dst2,
      b1.reshape(1, 64))

    sc2 = pl.kernel(
        _sc2_body,
        out_type=jax.ShapeDtypeStruct((2 * 2 * N, 16), jnp.float32),
        mesh=mesh,
        compiler_params=pltpu.CompilerParams(use_tc_tiling_on_sc=False),
        scratch_types=sc_scratch,
    )
    acc2 = sc2(src, dst, t2fa, tad2, tas2)

    accT = acc2.reshape(4, bsz, NUM_NODES, 16).transpose(2, 0, 1, 3)
    out = pl.pallas_call(
        _head_kernel,
        grid=(NUM_NODES,),
        in_specs=[
            pl.BlockSpec((1, 4, bsz, 16), lambda n: (n, 0, 0, 0)),
            pl.BlockSpec((1, 16), lambda n: (0, 0)),
            pl.BlockSpec((1, 16, 16), lambda n: (n, 0, 0)),
            pl.BlockSpec((1, 1, 16), lambda n: (n, 0, 0)),
        ],
        out_specs=pl.BlockSpec((1, bsz, 16), lambda n: (n, 0, 0)),
        out_shape=jax.ShapeDtypeStruct((NUM_NODES, bsz, 16), jnp.float32),
    )(accT, b2.reshape(1, 16), head_W, head_b.reshape(NUM_NODES, 1, 16))
    return out.transpose(1, 0, 2)


# speedup vs baseline: 37.6109x; 1.5715x over previous
import jax
import jax.numpy as jnp
from jax import lax
from jax.experimental import pallas as pl
from jax.experimental.pallas import tpu as pltpu
from jax.experimental.pallas import tpu_sc as plsc

NUM_NODES = 100
NV = 17
NCLS = NUM_NODES * NV
NCLSP = 1792
N = 102400
E = 1228800
NC, NS = 2, 16

E2 = E // NC
EPT = E2 // NS
KB = 128
NBLK = EPT // KB
HALF = N // NC
NPTH = HALF // NS
SB = 128
NPT = N // NS
ZB = 128


def _lrelu_exp(z):
    return jnp.exp(jnp.where(z >= 0, z, z * 0.2))


def _tables_kernel(emb_ref, w1_ref, as_ref, ad_ref, ct_ref):
    tb = jnp.dot(emb_ref[...], w1_ref[...], preferred_element_type=jnp.float32)
    cols = []
    for h in range(4):
        tbh = tb[:, 16 * h:16 * h + 16]
        a_s = jnp.sum(tbh * as_ref[h:h + 1, :], axis=1, keepdims=True)
        a_d = jnp.sum(tbh * ad_ref[h:h + 1, :], axis=1, keepdims=True)
        cols += [tbh, a_s, a_d]
    ct = jnp.concatenate(cols, axis=1)
    ct_ref[...] = jnp.concatenate(
        [ct, jnp.zeros((NCLSP - NCLS, 72), jnp.float32)], axis=0)


def _expand_kernel(x_ref, ct_ref, *out_refs):
    nb = x_ref.shape[0]
    gid = pl.program_id(0)
    ids = gid * nb + jax.lax.broadcasted_iota(jnp.int32, (nb, 1), 0)
    typ = ids - (ids // NUM_NODES) * NUM_NODES
    xv = jnp.clip(x_ref[...], 0, 16).astype(jnp.int32)
    cls = typ * NV + xv
    onehot = (cls == jax.lax.broadcasted_iota(jnp.int32, (nb, NCLSP), 1))
    res = jnp.dot(onehot.astype(jnp.float32), ct_ref[...],
                  preferred_element_type=jnp.float32)
    ones = jnp.ones((1, 16), jnp.float32)
    ones4 = jnp.ones((1, 4), jnp.float32)
    as4, ad4 = [], []
    for h in range(4):
        feat = res[:, 18 * h:18 * h + 16]
        asc = res[:, 18 * h + 16:18 * h + 17]
        adc = res[:, 18 * h + 17:18 * h + 18]
        out_refs[h][...] = jnp.concatenate([feat, asc * ones], axis=1)
        out_refs[4 + h][...] = adc * ones
        as4.append(asc * ones4)
        ad4.append(adc * ones4)
    out_refs[8][...] = jnp.concatenate(as4, axis=1)
    out_refs[9][...] = jnp.concatenate(ad4, axis=1)


def _zero_acc(acc_sh, outb, base):
    z16 = jnp.zeros((16,), jnp.float32)

    @pl.loop(0, ZB)
    def _(i):
        outb[i, pl.ds(0, 16)] = z16

    @pl.loop(0, NPT // ZB)
    def _(i):
        pltpu.sync_copy(outb, acc_sh.at[pl.ds(base + i * ZB, ZB)])


def _edge_pass_db(src_hbm, dst_hbm, tab_hbm, ad_hbm, acc_sh, bufs, eoff,
                  feat):
    srcb, dstb, tb, adr, outb, sdst, isem, osem = bufs

    def _issue_in(j, off):
        pltpu.async_copy(src_hbm.at[pl.ds(off, KB)], srcb[j], isem[j])
        pltpu.async_copy(dst_hbm.at[pl.ds(off, KB)], dstb[j], isem[j])

    def _issue_gather(j):
        pltpu.async_copy(tab_hbm.at[srcb[j]], tb[j], isem[j])
        pltpu.async_copy(ad_hbm.at[dstb[j]], adr[j], isem[j])

    def _wait_ids(j, off):
        pltpu.make_async_copy(src_hbm.at[pl.ds(off, KB)], srcb[j],
                              isem[j]).wait()
        pltpu.make_async_copy(dst_hbm.at[pl.ds(off, KB)], dstb[j],
                              isem[j]).wait()

    def _wait_gather(j):
        pltpu.make_async_copy(tab_hbm.at[srcb[j]], tb[j], isem[j]).wait()
        pltpu.make_async_copy(ad_hbm.at[dstb[j]], adr[j], isem[j]).wait()

    for j in range(2):
        _issue_in(j, eoff + j * KB)
    for j in range(2):
        _wait_ids(j, eoff + j * KB)
        _issue_gather(j)

    @pl.loop(0, NBLK // 2)
    def _(i):
        for j in range(2):
            b = 2 * i + j

            @pl.when(i > 0)
            def _():
                pltpu.make_async_copy(outb[j], acc_sh.at[sdst[j]],
                                      osem[j]).wait()

            _wait_gather(j)

            if feat:
                @pl.loop(0, KB, unroll=8)
                def _(r):
                    w = _lrelu_exp(tb[j][r, pl.ds(16, 16)]
                                   + adr[j][r, pl.ds(0, 16)])
                    outb[j][r, pl.ds(0, 16)] = tb[j][r, pl.ds(0, 16)] * w
            else:
                @pl.loop(0, KB, unroll=8)
                def _(r):
                    outb[j][r, pl.ds(0, 16)] = _lrelu_exp(
                        tb[j][r, pl.ds(0, 16)] + adr[j][r, pl.ds(0, 16)])

            @pl.loop(0, KB // 16)
            def _(g):
                sdst[j][pl.ds(g * 16, 16)] = dstb[j][pl.ds(g * 16, 16)]

            pltpu.async_copy(outb[j], acc_sh.at[sdst[j]], osem[j], add=True)

            @pl.when(b + 2 < NBLK)
            def _():
                _issue_in(j, eoff + (b + 2) * KB)
                _wait_ids(j, eoff + (b + 2) * KB)
                _issue_gather(j)

    for j in range(2):
        pltpu.make_async_copy(outb[j], acc_sh.at[sdst[j]], osem[j]).wait()


def _feat_self_pass(fa_hbm, ad_hbm, acc_sh, fab, adrows, outb, idxb, noff):
    @pl.loop(0, NPTH // SB)
    def _(b):
        goff = noff + b * SB
        pltpu.sync_copy(fa_hbm.at[pl.ds(goff, SB)], fab)
        pltpu.sync_copy(ad_hbm.at[pl.ds(goff, SB)], adrows)

        @pl.loop(0, SB // 16)
        def _(g):
            idxb[pl.ds(g * 16, 16)] = (jnp.arange(16, dtype=jnp.int32)
                                       + (goff + g * 16))

        @pl.loop(0, SB, unroll=8)
        def _(r):
            w = _lrelu_exp(fab[r, pl.ds(16, 16)] + adrows[r, pl.ds(0, 16)])
            outb[r, pl.ds(0, 16)] = fab[r, pl.ds(0, 16)] * w

        pltpu.sync_copy(outb, acc_sh.at[idxb], add=True)


def _den_self_pass(as_hbm, ad_hbm, acc_sh, asrows, adrows, outb, idxb, noff):
    @pl.loop(0, NPTH // SB)
    def _(b):
        goff = noff + b * SB
        pltpu.sync_copy(as_hbm.at[pl.ds(goff, SB)], asrows)
        pltpu.sync_copy(ad_hbm.at[pl.ds(goff, SB)], adrows)

        @pl.loop(0, SB // 16)
        def _(g):
            idxb[pl.ds(g * 16, 16)] = (jnp.arange(16, dtype=jnp.int32)
                                       + (goff + g * 16))

        @pl.loop(0, SB, unroll=8)
        def _(r):
            outb[r, pl.ds(0, 16)] = _lrelu_exp(asrows[r, pl.ds(0, 16)]
                                               + adrows[r, pl.ds(0, 16)])

        pltpu.sync_copy(outb, acc_sh.at[idxb], add=True)


def _writeback(acc_sh, acc_hbm, sid, slot_off):
    pltpu.sync_copy(acc_sh.at[pl.ds(sid * NPT, NPT)],
                    acc_hbm.at[pl.ds(slot_off + sid * NPT, NPT)])


def _sc1_body(src_hbm, dst_hbm, fa0, fa1, fa2, fa3, ad0, ad1, ad2, ad3,
              as4_hbm, ad4_hbm, acc_hbm,
              acc_sh, srcb0, srcb1, dstb0, dstb1, fab0, fab1, asr0, asr1,
              adr0, adr1, outb0, outb1, sdst0, sdst1, idxb,
              isem0, isem1, osem0, osem1):
    sid = lax.axis_index("s")
    cid = lax.axis_index("c")
    fas = [fa0, fa1, fa2, fa3]
    ads = [ad0, ad1, ad2, ad3]
    eoff = cid * E2 + sid * EPT
    noff = cid * HALF + sid * NPTH
    fbufs = ((srcb0, srcb1), (dstb0, dstb1), (fab0, fab1), (adr0, adr1),
             (outb0, outb1), (sdst0, sdst1), (isem0, isem1), (osem0, osem1))
    dbufs = ((srcb0, srcb1), (dstb0, dstb1), (asr0, asr1), (adr0, adr1),
             (outb0, outb1), (sdst0, sdst1), (isem0, isem1), (osem0, osem1))
    for p in range(4):
        _zero_acc(acc_sh, outb0, sid * NPT)
        plsc.subcore_barrier()
        _edge_pass_db(src_hbm, dst_hbm, fas[p], ads[p], acc_sh, fbufs, eoff,
                      True)
        _feat_self_pass(fas[p], ads[p], acc_sh, fab0, adr0, outb0, idxb,
                        noff)
        plsc.subcore_barrier()
        _writeback(acc_sh, acc_hbm, sid, cid * 5 * N + p * N)
        plsc.subcore_barrier()
    _zero_acc(acc_sh, outb0, sid * NPT)
    plsc.subcore_barrier()
    _edge_pass_db(src_hbm, dst_hbm, as4_hbm, ad4_hbm, acc_sh, dbufs, eoff,
                  False)
    _den_self_pass(as4_hbm, ad4_hbm, acc_sh, asr0, adr0, outb0, idxb, noff)
    plsc.subcore_barrier()
    _writeback(acc_sh, acc_hbm, sid, cid * 5 * N + 4 * N)


def _sc2_body(src_hbm, dst_hbm, fa_hbm, ad_hbm, as2_hbm, acc_hbm,
              acc_sh, srcb0, srcb1, dstb0, dstb1, fab0, fab1, asr0, asr1,
              adr0, adr1, outb0, outb1, sdst0, sdst1, idxb,
              isem0, isem1, osem0, osem1):
    sid = lax.axis_index("s")
    cid = lax.axis_index("c")
    eoff = cid * E2 + sid * EPT
    noff = cid * HALF + sid * NPTH
    fbufs = ((srcb0, srcb1), (dstb0, dstb1), (fab0, fab1), (adr0, adr1),
             (outb0, outb1), (sdst0, sdst1), (isem0, isem1), (osem0, osem1))
    dbufs = ((srcb0, srcb1), (dstb0, dstb1), (asr0, asr1), (adr0, adr1),
             (outb0, outb1), (sdst0, sdst1), (isem0, isem1), (osem0, osem1))
    _zero_acc(acc_sh, outb0, sid * NPT)
    plsc.subcore_barrier()
    _edge_pass_db(src_hbm, dst_hbm, fa_hbm, ad_hbm, acc_sh, fbufs, eoff,
                  True)
    _feat_self_pass(fa_hbm, ad_hbm, acc_sh, fab0, adr0, outb0, idxb, noff)
    plsc.subcore_barrier()
    _writeback(acc_sh, acc_hbm, sid, cid * 2 * N)
    plsc.subcore_barrier()
    _zero_acc(acc_sh, outb0, sid * NPT)
    plsc.subcore_barrier()
    _edge_pass_db(src_hbm, dst_hbm, as2_hbm, ad_hbm, acc_sh, dbufs, eoff,
                  False)
    _den_self_pass(as2_hbm, ad_hbm, acc_sh, asr0, adr0, outb0, idxb, noff)
    plsc.subcore_barrier()
    _writeback(acc_sh, acc_hbm, sid, cid * 2 * N + N)


def _mid_kernel(acc_ref, w2_ref, as2_ref, ad2_ref, b1_ref,
                fa_ref, ad_ref, as_ref):
    den = acc_ref[4] + acc_ref[9]
    outs = []
    for h in range(4):
        f = acc_ref[h] + acc_ref[5 + h]
        outs.append(f / (den[:, 4 * h:4 * h + 1] + 1e-16))
    x2 = jnp.maximum(jnp.concatenate(outs, axis=1) + b1_ref[...], 0.0)
    h2 = jnp.dot(x2, w2_ref[...], preferred_element_type=jnp.float32)
    ones = jnp.ones((1, 16), jnp.float32)
    a_s2 = jnp.sum(h2 * as2_ref[...], axis=1, keepdims=True)
    a_d2 = jnp.sum(h2 * ad2_ref[...], axis=1, keepdims=True)
    fa_ref[...] = jnp.concatenate([h2, a_s2 * ones], axis=1)
    ad_ref[...] = a_d2 * ones
    as_ref[...] = a_s2 * ones


def _head_kernel(acc_ref, b2_ref, w_ref, hb_ref, o_ref):
    f = acc_ref[0, 0] + acc_ref[0, 2]
    den = acc_ref[0, 1] + acc_ref[0, 3]
    hf = f / (den[:, 0:1] + 1e-16) + b2_ref[...]
    o_ref[0] = (jnp.dot(hf, w_ref[0], preferred_element_type=jnp.float32)
                + hb_ref[0])


def kernel(x, edge_index, emb_tables, W1, att_src1, att_dst1, b1, W2,
           att_src2, att_dst2, b2, head_W, head_b):
    bsz = x.shape[0] // NUM_NODES
    src = edge_index[0].astype(jnp.int32)
    dst = edge_index[1].astype(jnp.int32)

    ct = pl.pallas_call(
        _tables_kernel,
        out_shape=jax.ShapeDtypeStruct((NCLSP, 72), jnp.float32),
    )(emb_tables.reshape(NCLS, 16), W1, att_src1, att_dst1)

    nb = 512
    outs = pl.pallas_call(
        _expand_kernel,
        grid=(N // nb,),
        in_specs=[
            pl.BlockSpec((nb, 1), lambda g: (g, 0)),
            pl.BlockSpec((NCLSP, 72), lambda g: (0, 0)),
        ],
        out_specs=tuple([pl.BlockSpec((nb, 32), lambda g: (g, 0))] * 4
                        + [pl.BlockSpec((nb, 16), lambda g: (g, 0))] * 6),
        out_shape=tuple([jax.ShapeDtypeStruct((N, 32), jnp.float32)] * 4
                        + [jax.ShapeDtypeStruct((N, 16), jnp.float32)] * 6),
    )(x, ct)
    fa_tabs, ad_tabs, as4, ad4 = outs[:4], outs[4:8], outs[8], outs[9]

    mesh = plsc.VectorSubcoreMesh(core_axis_name="c", subcore_axis_name="s",
                                  num_cores=NC, num_subcores=NS)
    sc_scratch = (
        [pltpu.VMEM_SHARED((N, 16), jnp.float32)]
        + [pltpu.VMEM((KB,), jnp.int32)] * 4
        + [pltpu.VMEM((KB, 32), jnp.float32)] * 2
        + [pltpu.VMEM((KB, 16), jnp.float32)] * 6
        + [pltpu.VMEM((KB,), jnp.int32)] * 3
        + [pltpu.SemaphoreType.DMA] * 4
    )

    sc1 = pl.kernel(
        _sc1_body,
        out_type=jax.ShapeDtypeStruct((2 * 5 * N, 16), jnp.float32),
        mesh=mesh,
        compiler_params=pltpu.CompilerParams(use_tc_tiling_on_sc=False),
        scratch_types=sc_scratch,
    )
    acc1 = sc1(src, dst, *fa_tabs, *ad_tabs, as4, ad4)

    t2fa, tad2, tas2 = pl.pallas_call(
        _mid_kernel,
        grid=(N // 1024,),
        in_specs=[
            pl.BlockSpec((10, 1024, 16), lambda g: (0, g, 0)),
            pl.BlockSpec((64, 16), lambda g: (0, 0)),
            pl.BlockSpec((1, 16), lambda g: (0, 0)),
            pl.BlockSpec((1, 16), lambda g: (0, 0)),
            pl.BlockSpec((1, 64), lambda g: (0, 0)),
        ],
        out_specs=(pl.BlockSpec((1024, 32), lambda g: (g, 0)),
                   pl.BlockSpec((1024, 16), lambda g: (g, 0)),
                   pl.BlockSpec((1024, 16), lambda g: (g, 0))),
        out_shape=(jax.ShapeDtypeStruct((N, 32), jnp.float32),
                   jax.ShapeDtypeStruct((N, 16), jnp.float32),
                   jax.ShapeDtypeStruct((N, 16), jnp.float32)),
    )(acc1.reshape(2, 5, N, 16).reshape(10, N, 16), W2, att_src2, att_dst2,
      b1.reshape(1, 64))

    sc2 = pl.kernel(
        _sc2_body,
        out_type=jax.ShapeDtypeStruct((2 * 2 * N, 16), jnp.float32),
        mesh=mesh,
        compiler_params=pltpu.CompilerParams(use_tc_tiling_on_sc=False),
        scratch_types=sc_scratch,
    )
    acc2 = sc2(src, dst, t2fa, tad2, tas2)

    accT = acc2.reshape(4, bsz, NUM_NODES, 16).transpose(2, 0, 1, 3)
    out = pl.pallas_call(
        _head_kernel,
        grid=(NUM_NODES,),
        in_specs=[
            pl.BlockSpec((1, 4, bsz, 16), lambda n: (n, 0, 0, 0)),
            pl.BlockSpec((1, 16), lambda n: (0, 0)),
            pl.BlockSpec((1, 16, 16), lambda n: (n, 0, 0)),
            pl.BlockSpec((1, 1, 16), lambda n: (n, 0, 0)),
        ],
        out_specs=pl.BlockSpec((1, bsz, 16), lambda n: (n, 0, 0)),
        out_shape=jax.ShapeDtypeStruct((NUM_NODES, bsz, 16), jnp.float32),
    )(accT, b2.reshape(1, 16), head_W, head_b.reshape(NUM_NODES, 1, 16))
    return out.transpose(1, 0, 2)


# speedup vs baseline: 43.5605x; 1.1582x over previous
import jax
import jax.numpy as jnp
from jax import lax
from jax.experimental import pallas as pl
from jax.experimental.pallas import tpu as pltpu
from jax.experimental.pallas import tpu_sc as plsc

NUM_NODES = 100
NV = 17
NCLS = NUM_NODES * NV
NCLSP = 1792
N = 102400
E = 1228800
NC, NS = 2, 16

E2 = E // NC
EPT = E2 // NS
KB = 128
NBLK = EPT // KB
HALF = N // NC
NPTH = HALF // NS
SB = 128
NPT = N // NS
ZB = 128


def _lrelu_exp(z):
    return jnp.exp(jnp.where(z >= 0, z, z * 0.2))


def _tables_kernel(emb_ref, w1_ref, as_ref, ad_ref, ct_ref):
    tb = jnp.dot(emb_ref[...], w1_ref[...], preferred_element_type=jnp.float32)
    cols = []
    for h in range(4):
        tbh = tb[:, 16 * h:16 * h + 16]
        a_s = jnp.sum(tbh * as_ref[h:h + 1, :], axis=1, keepdims=True)
        a_d = jnp.sum(tbh * ad_ref[h:h + 1, :], axis=1, keepdims=True)
        cols += [tbh, a_s, a_d]
    ct = jnp.concatenate(cols, axis=1)
    ct_ref[...] = jnp.concatenate(
        [ct, jnp.zeros((NCLSP - NCLS, 72), jnp.float32)], axis=0)


def _expand_kernel(x_ref, ct_ref, *out_refs):
    nb = x_ref.shape[0]
    gid = pl.program_id(0)
    ids = gid * nb + jax.lax.broadcasted_iota(jnp.int32, (nb, 1), 0)
    typ = ids - (ids // NUM_NODES) * NUM_NODES
    xv = jnp.clip(x_ref[...], 0, 16).astype(jnp.int32)
    cls = typ * NV + xv
    onehot = (cls == jax.lax.broadcasted_iota(jnp.int32, (nb, NCLSP), 1))
    res = jnp.dot(onehot.astype(jnp.float32), ct_ref[...],
                  preferred_element_type=jnp.float32)
    ones = jnp.ones((1, 16), jnp.float32)
    ones4 = jnp.ones((1, 4), jnp.float32)
    as4, ad4 = [], []
    for h in range(4):
        feat = res[:, 18 * h:18 * h + 16]
        asc = res[:, 18 * h + 16:18 * h + 17]
        adc = res[:, 18 * h + 17:18 * h + 18]
        out_refs[h][...] = jnp.concatenate([feat, asc * ones], axis=1)
        out_refs[4 + h][...] = adc * ones
        as4.append(asc * ones4)
        ad4.append(adc * ones4)
    out_refs[8][...] = jnp.concatenate(as4, axis=1)
    out_refs[9][...] = jnp.concatenate(ad4, axis=1)


def _zero_acc(acc_sh, outb, base):
    z16 = jnp.zeros((16,), jnp.float32)

    @pl.loop(0, ZB)
    def _(i):
        outb[i, pl.ds(0, 16)] = z16

    @pl.loop(0, NPT // ZB)
    def _(i):
        pltpu.sync_copy(outb, acc_sh.at[pl.ds(base + i * ZB, ZB)])


def _edge_pass_db(src_hbm, dst_hbm, tab_hbm, ad_hbm, acc_sh, bufs, eoff,
                  feat):
    srcb, dstb, tb, adr, outb, sdst, isem, osem = bufs

    def _issue_in(j, off):
        pltpu.async_copy(src_hbm.at[pl.ds(off, KB)], srcb[j], isem[j])
        pltpu.async_copy(dst_hbm.at[pl.ds(off, KB)], dstb[j], isem[j])

    def _issue_gather(j):
        pltpu.async_copy(tab_hbm.at[srcb[j]], tb[j], isem[j])
        pltpu.async_copy(ad_hbm.at[dstb[j]], adr[j], isem[j])

    def _wait_ids(j, off):
        pltpu.make_async_copy(src_hbm.at[pl.ds(off, KB)], srcb[j],
                              isem[j]).wait()
        pltpu.make_async_copy(dst_hbm.at[pl.ds(off, KB)], dstb[j],
                              isem[j]).wait()

    def _wait_gather(j):
        pltpu.make_async_copy(tab_hbm.at[srcb[j]], tb[j], isem[j]).wait()
        pltpu.make_async_copy(ad_hbm.at[dstb[j]], adr[j], isem[j]).wait()

    for j in range(2):
        _issue_in(j, eoff + j * KB)
    for j in range(2):
        _wait_ids(j, eoff + j * KB)
        _issue_gather(j)

    @pl.loop(0, NBLK // 2)
    def _(i):
        for j in range(2):
            b = 2 * i + j

            @pl.when(i > 0)
            def _():
                pltpu.make_async_copy(outb[j], acc_sh.at[sdst[j]],
                                      osem[j]).wait()

            _wait_gather(j)

            @pl.loop(0, KB // 16)
            def _(g):
                sdst[j][pl.ds(g * 16, 16)] = dstb[j][pl.ds(g * 16, 16)]

            @pl.when(b + 2 < NBLK)
            def _():
                _issue_in(j, eoff + (b + 2) * KB)

            if feat:
                @pl.loop(0, KB, unroll=8)
                def _(r):
                    w = _lrelu_exp(tb[j][r, pl.ds(16, 16)]
                                   + adr[j][r, pl.ds(0, 16)])
                    outb[j][r, pl.ds(0, 16)] = tb[j][r, pl.ds(0, 16)] * w
            else:
                @pl.loop(0, KB, unroll=8)
                def _(r):
                    outb[j][r, pl.ds(0, 16)] = _lrelu_exp(
                        tb[j][r, pl.ds(0, 16)] + adr[j][r, pl.ds(0, 16)])

            pltpu.async_copy(outb[j], acc_sh.at[sdst[j]], osem[j], add=True)

            @pl.when(b + 2 < NBLK)
            def _():
                _wait_ids(j, eoff + (b + 2) * KB)
                _issue_gather(j)

    for j in range(2):
        pltpu.make_async_copy(outb[j], acc_sh.at[sdst[j]], osem[j]).wait()


def _feat_self_pass(fa_hbm, ad_hbm, acc_sh, fab, adrows, outb, idxb, noff):
    @pl.loop(0, NPTH // SB)
    def _(b):
        goff = noff + b * SB
        pltpu.sync_copy(fa_hbm.at[pl.ds(goff, SB)], fab)
        pltpu.sync_copy(ad_hbm.at[pl.ds(goff, SB)], adrows)

        @pl.loop(0, SB // 16)
        def _(g):
            idxb[pl.ds(g * 16, 16)] = (jnp.arange(16, dtype=jnp.int32)
                                       + (goff + g * 16))

        @pl.loop(0, SB, unroll=8)
        def _(r):
            w = _lrelu_exp(fab[r, pl.ds(16, 16)] + adrows[r, pl.ds(0, 16)])
            outb[r, pl.ds(0, 16)] = fab[r, pl.ds(0, 16)] * w

        pltpu.sync_copy(outb, acc_sh.at[idxb], add=True)


def _den_self_pass(as_hbm, ad_hbm, acc_sh, asrows, adrows, outb, idxb, noff):
    @pl.loop(0, NPTH // SB)
    def _(b):
        goff = noff + b * SB
        pltpu.sync_copy(as_hbm.at[pl.ds(goff, SB)], asrows)
        pltpu.sync_copy(ad_hbm.at[pl.ds(goff, SB)], adrows)

        @pl.loop(0, SB // 16)
        def _(g):
            idxb[pl.ds(g * 16, 16)] = (jnp.arange(16, dtype=jnp.int32)
                                       + (goff + g * 16))

        @pl.loop(0, SB, unroll=8)
        def _(r):
            outb[r, pl.ds(0, 16)] = _lrelu_exp(asrows[r, pl.ds(0, 16)]
                                               + adrows[r, pl.ds(0, 16)])

        pltpu.sync_copy(outb, acc_sh.at[idxb], add=True)


def _writeback(acc_sh, acc_hbm, sid, slot_off):
    pltpu.sync_copy(acc_sh.at[pl.ds(sid * NPT, NPT)],
                    acc_hbm.at[pl.ds(slot_off + sid * NPT, NPT)])


def _sc1_body(src_hbm, dst_hbm, fa0, fa1, fa2, fa3, ad0, ad1, ad2, ad3,
              as4_hbm, ad4_hbm, acc_hbm,
              acc_sh, srcb0, srcb1, dstb0, dstb1, fab0, fab1, asr0, asr1,
              adr0, adr1, outb0, outb1, sdst0, sdst1, idxb,
              isem0, isem1, osem0, osem1):
    sid = lax.axis_index("s")
    cid = lax.axis_index("c")
    fas = [fa0, fa1, fa2, fa3]
    ads = [ad0, ad1, ad2, ad3]
    eoff = cid * E2 + sid * EPT
    noff = cid * HALF + sid * NPTH
    fbufs = ((srcb0, srcb1), (dstb0, dstb1), (fab0, fab1), (adr0, adr1),
             (outb0, outb1), (sdst0, sdst1), (isem0, isem1), (osem0, osem1))
    dbufs = ((srcb0, srcb1), (dstb0, dstb1), (asr0, asr1), (adr0, adr1),
             (outb0, outb1), (sdst0, sdst1), (isem0, isem1), (osem0, osem1))
    for p in range(4):
        _zero_acc(acc_sh, outb0, sid * NPT)
        plsc.subcore_barrier()
        _edge_pass_db(src_hbm, dst_hbm, fas[p], ads[p], acc_sh, fbufs, eoff,
                      True)
        _feat_self_pass(fas[p], ads[p], acc_sh, fab0, adr0, outb0, idxb,
                        noff)
        plsc.subcore_barrier()
        _writeback(acc_sh, acc_hbm, sid, cid * 5 * N + p * N)
        plsc.subcore_barrier()
    _zero_acc(acc_sh, outb0, sid * NPT)
    plsc.subcore_barrier()
    _edge_pass_db(src_hbm, dst_hbm, as4_hbm, ad4_hbm, acc_sh, dbufs, eoff,
                  False)
    _den_self_pass(as4_hbm, ad4_hbm, acc_sh, asr0, adr0, outb0, idxb, noff)
    plsc.subcore_barrier()
    _writeback(acc_sh, acc_hbm, sid, cid * 5 * N + 4 * N)


def _sc2_body(src_hbm, dst_hbm, fa_hbm, ad_hbm, as2_hbm, acc_hbm,
              acc_sh, srcb0, srcb1, dstb0, dstb1, fab0, fab1, asr0, asr1,
              adr0, adr1, outb0, outb1, sdst0, sdst1, idxb,
              isem0, isem1, osem0, osem1):
    sid = lax.axis_index("s")
    cid = lax.axis_index("c")
    eoff = cid * E2 + sid * EPT
    noff = cid * HALF + sid * NPTH
    fbufs = ((srcb0, srcb1), (dstb0, dstb1), (fab0, fab1), (adr0, adr1),
             (outb0, outb1), (sdst0, sdst1), (isem0, isem1), (osem0, osem1))
    dbufs = ((srcb0, srcb1), (dstb0, dstb1), (asr0, asr1), (adr0, adr1),
             (outb0, outb1), (sdst0, sdst1), (isem0, isem1), (osem0, osem1))
    _zero_acc(acc_sh, outb0, sid * NPT)
    plsc.subcore_barrier()
    _edge_pass_db(src_hbm, dst_hbm, fa_hbm, ad_hbm, acc_sh, fbufs, eoff,
                  True)
    _feat_self_pass(fa_hbm, ad_hbm, acc_sh, fab0, adr0, outb0, idxb, noff)
    plsc.subcore_barrier()
    _writeback(acc_sh, acc_hbm, sid, cid * 2 * N)
    plsc.subcore_barrier()
    _zero_acc(acc_sh, outb0, sid * NPT)
    plsc.subcore_barrier()
    _edge_pass_db(src_hbm, dst_hbm, as2_hbm, ad_hbm, acc_sh, dbufs, eoff,
                  False)
    _den_self_pass(as2_hbm, ad_hbm, acc_sh, asr0, adr0, outb0, idxb, noff)
    plsc.subcore_barrier()
    _writeback(acc_sh, acc_hbm, sid, cid * 2 * N + N)


def _mid_kernel(acc_ref, w2_ref, as2_ref, ad2_ref, b1_ref,
                fa_ref, ad_ref, as_ref):
    den = acc_ref[4] + acc_ref[9]
    outs = []
    for h in range(4):
        f = acc_ref[h] + acc_ref[5 + h]
        outs.append(f / (den[:, 4 * h:4 * h + 1] + 1e-16))
    x2 = jnp.maximum(jnp.concatenate(outs, axis=1) + b1_ref[...], 0.0)
    h2 = jnp.dot(x2, w2_ref[...], preferred_element_type=jnp.float32)
    ones = jnp.ones((1, 16), jnp.float32)
    a_s2 = jnp.sum(h2 * as2_ref[...], axis=1, keepdims=True)
    a_d2 = jnp.sum(h2 * ad2_ref[...], axis=1, keepdims=True)
    fa_ref[...] = jnp.concatenate([h2, a_s2 * ones], axis=1)
    ad_ref[...] = a_d2 * ones
    as_ref[...] = a_s2 * ones


def _head_kernel(acc_ref, b2_ref, w_ref, hb_ref, o_ref):
    f = acc_ref[0, 0] + acc_ref[0, 2]
    den = acc_ref[0, 1] + acc_ref[0, 3]
    hf = f / (den[:, 0:1] + 1e-16) + b2_ref[...]
    o_ref[0] = (jnp.dot(hf, w_ref[0], preferred_element_type=jnp.float32)
                + hb_ref[0])


def kernel(x, edge_index, emb_tables, W1, att_src1, att_dst1, b1, W2,
           att_src2, att_dst2, b2, head_W, head_b):
    bsz = x.shape[0] // NUM_NODES
    src = edge_index[0].astype(jnp.int32)
    dst = edge_index[1].astype(jnp.int32)

    ct = pl.pallas_call(
        _tables_kernel,
        out_shape=jax.ShapeDtypeStruct((NCLSP, 72), jnp.float32),
    )(emb_tables.reshape(NCLS, 16), W1, att_src1, att_dst1)

    nb = 512
    outs = pl.pallas_call(
        _expand_kernel,
        grid=(N // nb,),
        in_specs=[
            pl.BlockSpec((nb, 1), lambda g: (g, 0)),
            pl.BlockSpec((NCLSP, 72), lambda g: (0, 0)),
        ],
        out_specs=tuple([pl.BlockSpec((nb, 32), lambda g: (g, 0))] * 4
                        + [pl.BlockSpec((nb, 16), lambda g: (g, 0))] * 6),
        out_shape=tuple([jax.ShapeDtypeStruct((N, 32), jnp.float32)] * 4
                        + [jax.ShapeDtypeStruct((N, 16), jnp.float32)] * 6),
    )(x, ct)
    fa_tabs, ad_tabs, as4, ad4 = outs[:4], outs[4:8], outs[8], outs[9]

    mesh = plsc.VectorSubcoreMesh(core_axis_name="c", subcore_axis_name="s",
                                  num_cores=NC, num_subcores=NS)
    sc_scratch = (
        [pltpu.VMEM_SHARED((N, 16), jnp.float32)]
        + [pltpu.VMEM((KB,), jnp.int32)] * 4
        + [pltpu.VMEM((KB, 32), jnp.float32)] * 2
        + [pltpu.VMEM((KB, 16), jnp.float32)] * 6
        + [pltpu.VMEM((KB,), jnp.int32)] * 3
        + [pltpu.SemaphoreType.DMA] * 4
    )

    sc1 = pl.kernel(
        _sc1_body,
        out_type=jax.ShapeDtypeStruct((2 * 5 * N, 16), jnp.float32),
        mesh=mesh,
        compiler_params=pltpu.CompilerParams(use_tc_tiling_on_sc=False),
        scratch_types=sc_scratch,
    )
    acc1 = sc1(src, dst, *fa_tabs, *ad_tabs, as4, ad4)

    t2fa, tad2, tas2 = pl.pallas_call(
        _mid_kernel,
        grid=(N // 1024,),
        in_specs=[
            pl.BlockSpec((10, 1024, 16), lambda g: (0, g, 0)),
            pl.BlockSpec((64, 16), lambda g: (0, 0)),
            pl.BlockSpec((1, 16), lambda g: (0, 0)),
            pl.BlockSpec((1, 16), lambda g: (0, 0)),
            pl.BlockSpec((1, 64), lambda g: (0, 0)),
        ],
        out_specs=(pl.BlockSpec((1024, 32), lambda g: (g, 0)),
                   pl.BlockSpec((1024, 16), lambda g: (g, 0)),
                   pl.BlockSpec((1024, 16), lambda g: (g, 0))),
        out_shape=(jax.ShapeDtypeStruct((N, 32), jnp.float32),
                   jax.ShapeDtypeStruct((N, 16), jnp.float32),
                   jax.ShapeDtypeStruct((N, 16), jnp.float32)),
    )(acc1.reshape(2, 5, N, 16).reshape(10, N, 16), W2, att_src2, att_dst2,
      b1.reshape(1, 64))

    sc2 = pl.kernel(
        _sc2_body,
        out_type=jax.ShapeDtypeStruct((2 * 2 * N, 16), jnp.float32),
        mesh=mesh,
        compiler_params=pltpu.CompilerParams(use_tc_tiling_on_sc=False),
        scratch_types=sc_scratch,
    )
    acc2 = sc2(src, dst, t2fa, tad2, tas2)

    accT = acc2.reshape(4, bsz, NUM_NODES, 16).transpose(2, 0, 1, 3)
    out = pl.pallas_call(
        _head_kernel,
        grid=(NUM_NODES,),
        in_specs=[
            pl.BlockSpec((1, 4, bsz, 16), lambda n: (n, 0, 0, 0)),
            pl.BlockSpec((1, 16), lambda n: (0, 0)),
            pl.BlockSpec((1, 16, 16), lambda n: (n, 0, 0)),
            pl.BlockSpec((1, 1, 16), lambda n: (n, 0, 0)),
        ],
        out_specs=pl.BlockSpec((1, bsz, 16), lambda n: (n, 0, 0)),
        out_shape=jax.ShapeDtypeStruct((NUM_NODES, bsz, 16), jnp.float32),
    )(accT, b2.reshape(1, 16), head_W, head_b.reshape(NUM_NODES, 1, 16))
    return out.transpose(1, 0, 2)


# speedup vs baseline: 43.6237x; 1.0015x over previous
import jax
import jax.numpy as jnp
from jax import lax
from jax.experimental import pallas as pl
from jax.experimental.pallas import tpu as pltpu
from jax.experimental.pallas import tpu_sc as plsc

NUM_NODES = 100
NV = 17
NCLS = NUM_NODES * NV
NCLSP = 1792
N = 102400
E = 1228800
NC, NS = 2, 16

E2 = E // NC
EPT = E2 // NS
KB = 128
NBLK = EPT // KB
HALF = N // NC
NPTH = HALF // NS
SB = 128
NPT = N // NS
ZB = 128


def _lrelu_exp(z):
    return jnp.exp(jnp.where(z >= 0, z, z * 0.2))


def _tables_kernel(emb_ref, w1_ref, as_ref, ad_ref, ct_ref):
    tb = jnp.dot(emb_ref[...], w1_ref[...], preferred_element_type=jnp.float32)
    cols = []
    for h in range(4):
        tbh = tb[:, 16 * h:16 * h + 16]
        a_s = jnp.sum(tbh * as_ref[h:h + 1, :], axis=1, keepdims=True)
        a_d = jnp.sum(tbh * ad_ref[h:h + 1, :], axis=1, keepdims=True)
        cols += [tbh, a_s, a_d]
    ct = jnp.concatenate(cols, axis=1)
    ct_ref[...] = jnp.concatenate(
        [ct, jnp.zeros((NCLSP - NCLS, 72), jnp.float32)], axis=0)


def _expand_kernel(x_ref, ct_ref, *out_refs):
    nb = x_ref.shape[0]
    gid = pl.program_id(0)
    ids = gid * nb + jax.lax.broadcasted_iota(jnp.int32, (nb, 1), 0)
    typ = ids - (ids // NUM_NODES) * NUM_NODES
    xv = jnp.clip(x_ref[...], 0, 16).astype(jnp.int32)
    cls = typ * NV + xv
    onehot = (cls == jax.lax.broadcasted_iota(jnp.int32, (nb, NCLSP), 1))
    res = jnp.dot(onehot.astype(jnp.float32), ct_ref[...],
                  preferred_element_type=jnp.float32)
    ones = jnp.ones((1, 16), jnp.float32)
    ones4 = jnp.ones((1, 4), jnp.float32)
    as4, ad4 = [], []
    for h in range(4):
        feat = res[:, 18 * h:18 * h + 16]
        asc = res[:, 18 * h + 16:18 * h + 17]
        adc = res[:, 18 * h + 17:18 * h + 18]
        out_refs[h][...] = jnp.concatenate([feat, asc * ones], axis=1)
        out_refs[4 + h][...] = adc * ones
        as4.append(asc * ones4)
        ad4.append(adc * ones4)
    out_refs[8][...] = jnp.concatenate(as4, axis=1)
    out_refs[9][...] = jnp.concatenate(ad4, axis=1)


def _zero_acc(acc_sh, outb, base):
    z16 = jnp.zeros((16,), jnp.float32)

    @pl.loop(0, ZB)
    def _(i):
        outb[i, pl.ds(0, 16)] = z16

    @pl.loop(0, NPT // ZB)
    def _(i):
        pltpu.sync_copy(outb, acc_sh.at[pl.ds(base + i * ZB, ZB)])


def _edge_pass_db(src_hbm, dst_hbm, tab_hbm, ad_hbm, acc_sh, bufs, eoff,
                  feat):
    srcb, dstb, tb, adr, outb, sdst, isem, osem = bufs

    def _issue_in(j, off):
        pltpu.async_copy(src_hbm.at[pl.ds(off, KB)], srcb[j], isem[j])
        pltpu.async_copy(dst_hbm.at[pl.ds(off, KB)], dstb[j], isem[j])

    def _issue_gather(j):
        pltpu.async_copy(tab_hbm.at[srcb[j]], tb[j], isem[j])
        pltpu.async_copy(ad_hbm.at[dstb[j]], adr[j], isem[j])

    def _wait_ids(j, off):
        pltpu.make_async_copy(src_hbm.at[pl.ds(off, KB)], srcb[j],
                              isem[j]).wait()
        pltpu.make_async_copy(dst_hbm.at[pl.ds(off, KB)], dstb[j],
                              isem[j]).wait()

    def _wait_gather(j):
        pltpu.make_async_copy(tab_hbm.at[srcb[j]], tb[j], isem[j]).wait()
        pltpu.make_async_copy(ad_hbm.at[dstb[j]], adr[j], isem[j]).wait()

    for j in range(2):
        _issue_in(j, eoff + j * KB)
    for j in range(2):
        _wait_ids(j, eoff + j * KB)
        _issue_gather(j)

    @pl.loop(0, NBLK // 2)
    def _(i):
        for j in range(2):
            b = 2 * i + j

            @pl.when(i > 0)
            def _():
                pltpu.make_async_copy(outb[j], acc_sh.at[sdst[j]],
                                      osem[j]).wait()

            _wait_gather(j)

            @pl.loop(0, KB // 16)
            def _(g):
                sdst[j][pl.ds(g * 16, 16)] = dstb[j][pl.ds(g * 16, 16)]

            @pl.when(b + 2 < NBLK)
            def _():
                _issue_in(j, eoff + (b + 2) * KB)

            if feat:
                @pl.loop(0, KB, unroll=16)
                def _(r):
                    w = _lrelu_exp(tb[j][r, pl.ds(16, 16)]
                                   + adr[j][r, pl.ds(0, 16)])
                    outb[j][r, pl.ds(0, 16)] = tb[j][r, pl.ds(0, 16)] * w
            else:
                @pl.loop(0, KB, unroll=16)
                def _(r):
                    outb[j][r, pl.ds(0, 16)] = _lrelu_exp(
                        tb[j][r, pl.ds(0, 16)] + adr[j][r, pl.ds(0, 16)])

            pltpu.async_copy(outb[j], acc_sh.at[sdst[j]], osem[j], add=True)

            @pl.when(b + 2 < NBLK)
            def _():
                _wait_ids(j, eoff + (b + 2) * KB)
                _issue_gather(j)

    for j in range(2):
        pltpu.make_async_copy(outb[j], acc_sh.at[sdst[j]], osem[j]).wait()


def _feat_self_pass(fa_hbm, ad_hbm, acc_sh, fab, adrows, outb, idxb, noff):
    @pl.loop(0, NPTH // SB)
    def _(b):
        goff = noff + b * SB
        pltpu.sync_copy(fa_hbm.at[pl.ds(goff, SB)], fab)
        pltpu.sync_copy(ad_hbm.at[pl.ds(goff, SB)], adrows)

        @pl.loop(0, SB // 16)
        def _(g):
            idxb[pl.ds(g * 16, 16)] = (jnp.arange(16, dtype=jnp.int32)
                                       + (goff + g * 16))

        @pl.loop(0, SB, unroll=16)
        def _(r):
            w = _lrelu_exp(fab[r, pl.ds(16, 16)] + adrows[r, pl.ds(0, 16)])
            outb[r, pl.ds(0, 16)] = fab[r, pl.ds(0, 16)] * w

        pltpu.sync_copy(outb, acc_sh.at[idxb], add=True)


def _den_self_pass(as_hbm, ad_hbm, acc_sh, asrows, adrows, outb, idxb, noff):
    @pl.loop(0, NPTH // SB)
    def _(b):
        goff = noff + b * SB
        pltpu.sync_copy(as_hbm.at[pl.ds(goff, SB)], asrows)
        pltpu.sync_copy(ad_hbm.at[pl.ds(goff, SB)], adrows)

        @pl.loop(0, SB // 16)
        def _(g):
            idxb[pl.ds(g * 16, 16)] = (jnp.arange(16, dtype=jnp.int32)
                                       + (goff + g * 16))

        @pl.loop(0, SB, unroll=16)
        def _(r):
            outb[r, pl.ds(0, 16)] = _lrelu_exp(asrows[r, pl.ds(0, 16)]
                                               + adrows[r, pl.ds(0, 16)])

        pltpu.sync_copy(outb, acc_sh.at[idxb], add=True)


def _writeback(acc_sh, acc_hbm, sid, slot_off):
    pltpu.sync_copy(acc_sh.at[pl.ds(sid * NPT, NPT)],
                    acc_hbm.at[pl.ds(slot_off + sid * NPT, NPT)])


def _sc1_body(src_hbm, dst_hbm, fa0, fa1, fa2, fa3, ad0, ad1, ad2, ad3,
              as4_hbm, ad4_hbm, acc_hbm,
              acc_sh, srcb0, srcb1, dstb0, dstb1, fab0, fab1, asr0, asr1,
              adr0, adr1, outb0, outb1, sdst0, sdst1, idxb,
              isem0, isem1, osem0, osem1):
    sid = lax.axis_index("s")
    cid = lax.axis_index("c")
    fas = [fa0, fa1, fa2, fa3]
    ads = [ad0, ad1, ad2, ad3]
    eoff = cid * E2 + sid * EPT
    noff = cid * HALF + sid * NPTH
    fbufs = ((srcb0, srcb1), (dstb0, dstb1), (fab0, fab1), (adr0, adr1),
             (outb0, outb1), (sdst0, sdst1), (isem0, isem1), (osem0, osem1))
    dbufs = ((srcb0, srcb1), (dstb0, dstb1), (asr0, asr1), (adr0, adr1),
             (outb0, outb1), (sdst0, sdst1), (isem0, isem1), (osem0, osem1))
    for p in range(4):
        _zero_acc(acc_sh, outb0, sid * NPT)
        plsc.subcore_barrier()
        _edge_pass_db(src_hbm, dst_hbm, fas[p], ads[p], acc_sh, fbufs, eoff,
                      True)
        _feat_self_pass(fas[p], ads[p], acc_sh, fab0, adr0, outb0, idxb,
                        noff)
        plsc.subcore_barrier()
        _writeback(acc_sh, acc_hbm, sid, cid * 5 * N + p * N)
        plsc.subcore_barrier()
    _zero_acc(acc_sh, outb0, sid * NPT)
    plsc.subcore_barrier()
    _edge_pass_db(src_hbm, dst_hbm, as4_hbm, ad4_hbm, acc_sh, dbufs, eoff,
                  False)
    _den_self_pass(as4_hbm, ad4_hbm, acc_sh, asr0, adr0, outb0, idxb, noff)
    plsc.subcore_barrier()
    _writeback(acc_sh, acc_hbm, sid, cid * 5 * N + 4 * N)


def _sc2_body(src_hbm, dst_hbm, fa_hbm, ad_hbm, as2_hbm, acc_hbm,
              acc_sh, srcb0, srcb1, dstb0, dstb1, fab0, fab1, asr0, asr1,
              adr0, adr1, outb0, outb1, sdst0, sdst1, idxb,
              isem0, isem1, osem0, osem1):
    sid = lax.axis_index("s")
    cid = lax.axis_index("c")
    eoff = cid * E2 + sid * EPT
    noff = cid * HALF + sid * NPTH
    fbufs = ((srcb0, srcb1), (dstb0, dstb1), (fab0, fab1), (adr0, adr1),
             (outb0, outb1), (sdst0, sdst1), (isem0, isem1), (osem0, osem1))
    dbufs = ((srcb0, srcb1), (dstb0, dstb1), (asr0, asr1), (adr0, adr1),
             (outb0, outb1), (sdst0, sdst1), (isem0, isem1), (osem0, osem1))
    _zero_acc(acc_sh, outb0, sid * NPT)
    plsc.subcore_barrier()
    _edge_pass_db(src_hbm, dst_hbm, fa_hbm, ad_hbm, acc_sh, fbufs, eoff,
                  True)
    _feat_self_pass(fa_hbm, ad_hbm, acc_sh, fab0, adr0, outb0, idxb, noff)
    plsc.subcore_barrier()
    _writeback(acc_sh, acc_hbm, sid, cid * 2 * N)
    plsc.subcore_barrier()
    _zero_acc(acc_sh, outb0, sid * NPT)
    plsc.subcore_barrier()
    _edge_pass_db(src_hbm, dst_hbm, as2_hbm, ad_hbm, acc_sh, dbufs, eoff,
                  False)
    _den_self_pass(as2_hbm, ad_hbm, acc_sh, asr0, adr0, outb0, idxb, noff)
    plsc.subcore_barrier()
    _writeback(acc_sh, acc_hbm, sid, cid * 2 * N + N)


def _mid_kernel(acc_ref, w2_ref, as2_ref, ad2_ref, b1_ref,
                fa_ref, ad_ref, as_ref):
    den = acc_ref[4] + acc_ref[9]
    outs = []
    for h in range(4):
        f = acc_ref[h] + acc_ref[5 + h]
        outs.append(f / (den[:, 4 * h:4 * h + 1] + 1e-16))
    x2 = jnp.maximum(jnp.concatenate(outs, axis=1) + b1_ref[...], 0.0)
    h2 = jnp.dot(x2, w2_ref[...], preferred_element_type=jnp.float32)
    ones = jnp.ones((1, 16), jnp.float32)
    a_s2 = jnp.sum(h2 * as2_ref[...], axis=1, keepdims=True)
    a_d2 = jnp.sum(h2 * ad2_ref[...], axis=1, keepdims=True)
    fa_ref[...] = jnp.concatenate([h2, a_s2 * ones], axis=1)
    ad_ref[...] = a_d2 * ones
    as_ref[...] = a_s2 * ones


def _head_kernel(acc_ref, b2_ref, w_ref, hb_ref, o_ref):
    f = acc_ref[0, 0] + acc_ref[0, 2]
    den = acc_ref[0, 1] + acc_ref[0, 3]
    hf = f / (den[:, 0:1] + 1e-16) + b2_ref[...]
    o_ref[0] = (jnp.dot(hf, w_ref[0], preferred_element_type=jnp.float32)
                + hb_ref[0])


def kernel(x, edge_index, emb_tables, W1, att_src1, att_dst1, b1, W2,
           att_src2, att_dst2, b2, head_W, head_b):
    bsz = x.shape[0] // NUM_NODES
    src = edge_index[0].astype(jnp.int32)
    dst = edge_index[1].astype(jnp.int32)

    ct = pl.pallas_call(
        _tables_kernel,
        out_shape=jax.ShapeDtypeStruct((NCLSP, 72), jnp.float32),
    )(emb_tables.reshape(NCLS, 16), W1, att_src1, att_dst1)

    nb = 512
    outs = pl.pallas_call(
        _expand_kernel,
        grid=(N // nb,),
        in_specs=[
            pl.BlockSpec((nb, 1), lambda g: (g, 0)),
            pl.BlockSpec((NCLSP, 72), lambda g: (0, 0)),
        ],
        out_specs=tuple([pl.BlockSpec((nb, 32), lambda g: (g, 0))] * 4
                        + [pl.BlockSpec((nb, 16), lambda g: (g, 0))] * 6),
        out_shape=tuple([jax.ShapeDtypeStruct((N, 32), jnp.float32)] * 4
                        + [jax.ShapeDtypeStruct((N, 16), jnp.float32)] * 6),
    )(x, ct)
    fa_tabs, ad_tabs, as4, ad4 = outs[:4], outs[4:8], outs[8], outs[9]

    mesh = plsc.VectorSubcoreMesh(core_axis_name="c", subcore_axis_name="s",
                                  num_cores=NC, num_subcores=NS)
    sc_scratch = (
        [pltpu.VMEM_SHARED((N, 16), jnp.float32)]
        + [pltpu.VMEM((KB,), jnp.int32)] * 4
        + [pltpu.VMEM((KB, 32), jnp.float32)] * 2
        + [pltpu.VMEM((KB, 16), jnp.float32)] * 6
        + [pltpu.VMEM((KB,), jnp.int32)] * 3
        + [pltpu.SemaphoreType.DMA] * 4
    )

    sc1 = pl.kernel(
        _sc1_body,
        out_type=jax.ShapeDtypeStruct((2 * 5 * N, 16), jnp.float32),
        mesh=mesh,
        compiler_params=pltpu.CompilerParams(use_tc_tiling_on_sc=False),
        scratch_types=sc_scratch,
    )
    acc1 = sc1(src, dst, *fa_tabs, *ad_tabs, as4, ad4)

    t2fa, tad2, tas2 = pl.pallas_call(
        _mid_kernel,
        grid=(N // 1024,),
        in_specs=[
            pl.BlockSpec((10, 1024, 16), lambda g: (0, g, 0)),
            pl.BlockSpec((64, 16), lambda g: (0, 0)),
            pl.BlockSpec((1, 16), lambda g: (0, 0)),
            pl.BlockSpec((1, 16), lambda g: (0, 0)),
            pl.BlockSpec((1, 64), lambda g: (0, 0)),
        ],
        out_specs=(pl.BlockSpec((1024, 32), lambda g: (g, 0)),
                   pl.BlockSpec((1024, 16), lambda g: (g, 0)),
                   pl.BlockSpec((1024, 16), lambda g: (g, 0))),
        out_shape=(jax.ShapeDtypeStruct((N, 32), jnp.float32),
                   jax.ShapeDtypeStruct((N, 16), jnp.float32),
                   jax.ShapeDtypeStruct((N, 16), jnp.float32)),
    )(acc1.reshape(2, 5, N, 16).reshape(10, N, 16), W2, att_src2, att_dst2,
      b1.reshape(1, 64))

    sc2 = pl.kernel(
        _sc2_body,
        out_type=jax.ShapeDtypeStruct((2 * 2 * N, 16), jnp.float32),
        mesh=mesh,
        compiler_params=pltpu.CompilerParams(use_tc_tiling_on_sc=False),
        scratch_types=sc_scratch,
    )
    acc2 = sc2(src, dst, t2fa, tad2, tas2)

    accT = acc2.reshape(4, bsz, NUM_NODES, 16).transpose(2, 0, 1, 3)
    out = pl.pallas_call(
        _head_kernel,
        grid=(NUM_NODES,),
        in_specs=[
            pl.BlockSpec((1, 4, bsz, 16), lambda n: (n, 0, 0, 0)),
            pl.BlockSpec((1, 16), lambda n: (0, 0)),
            pl.BlockSpec((1, 16, 16), lambda n: (n, 0, 0)),
            pl.BlockSpec((1, 1, 16), lambda n: (n, 0, 0)),
        ],
        out_specs=pl.BlockSpec((1, bsz, 16), lambda n: (n, 0, 0)),
        out_shape=jax.ShapeDtypeStruct((NUM_NODES, bsz, 16), jnp.float32),
    )(accT, b2.reshape(1, 16), head_W, head_b.reshape(NUM_NODES, 1, 16))
    return out.transpose(1, 0, 2)
